# Initial kernel scaffold; baseline (speedup 1.0000x reference)
import jax
import jax.numpy as jnp
from jax.experimental import pallas as pl


def kernel(x, edge_index, batch, bn_feat_g, bn_feat_b, Wf, bf, bn1_g, bn1_b, W1, b1, bn2_g, bn2_b, W2, b2, bn3_g, bn3_b, W3, b3, bn_fc_g, bn_fc_b, Wl, bl, bn_h_g, bn_h_b, Wc, bc):
    raise NotImplementedError("write your pallas kernel here")


# speedup vs baseline: 10.2246x; 10.2246x over previous
import functools

import jax
import jax.numpy as jnp
from jax import lax
from jax.experimental import pallas as pl
from jax.experimental.pallas import tpu as pltpu
from jax.experimental.pallas import tpu_sc as plsc

N = 10000
E = 320000
D = 128
H = 128
C = 10
G = 64

NSC = 2
NTILE = 16
NW = NSC * NTILE
EPW = E // NW
NP = 10240
RPT = NP // NTILE
K = 80
NCHUNK = EPW // K
ZR = 128
DEGW = 128

_MESH = plsc.VectorSubcoreMesh(core_axis_name="c", subcore_axis_name="s")


def _fill_f32(ref, rows, width, value):
    vec = jnp.full((16,), value, jnp.float32)

    def body(r, carry):
        for j in range(width // 16):
            ref[r, pl.ds(j * 16, 16)] = vec
        return carry

    lax.fori_loop(0, rows, body, 0)


@functools.partial(
    pl.kernel,
    out_type=jax.ShapeDtypeStruct((NSC * NP, DEGW), jnp.float32),
    mesh=_MESH,
    scratch_types=[
        pltpu.VMEM((K, DEGW), jnp.float32),
        pltpu.VMEM((K,), jnp.int32),
        pltpu.VMEM((ZR, DEGW), jnp.float32),
        pltpu.VMEM_SHARED((NP, DEGW), jnp.float32),
        pltpu.SemaphoreType.DMA,
    ],
)
def _sc_degree(row_hbm, out_hbm, ones_v, idx_v, zero_v, acc, sem):
    c = lax.axis_index("c")
    s = lax.axis_index("s")
    _fill_f32(ones_v, K, DEGW, 1.0)
    _fill_f32(zero_v, ZR, DEGW, 0.0)
    for t in range(RPT // ZR):
        pltpu.sync_copy(zero_v, acc.at[pl.ds(s * RPT + t * ZR, ZR), :])
    plsc.subcore_barrier()

    base = (c * NTILE + s) * EPW

    def body(i, carry):
        pltpu.sync_copy(row_hbm.at[pl.ds(base + i * K, K)], idx_v)
        pltpu.sync_copy(ones_v, acc.at[idx_v], add=True)
        return carry

    lax.fori_loop(0, NCHUNK, body, 0)
    plsc.subcore_barrier()
    pltpu.sync_copy(
        acc.at[pl.ds(s * RPT, RPT), :],
        out_hbm.at[pl.ds(c * NP + s * RPT, RPT), :],
    )


@functools.partial(
    pl.kernel,
    out_type=jax.ShapeDtypeStruct((NSC * NP, H), jnp.float32),
    mesh=_MESH,
    scratch_types=[
        pltpu.VMEM((K,), jnp.int32),
        pltpu.VMEM((K,), jnp.int32),
        pltpu.VMEM((K, H), jnp.float32),
        pltpu.VMEM((ZR, H), jnp.float32),
        pltpu.VMEM_SHARED((NP, H), jnp.float32),
        pltpu.SemaphoreType.DMA,
    ],
)
def _sc_aggregate(hp_hbm, row_hbm, col_hbm, out_hbm, col_v, row_v, rows_v, zero_v, acc, sem):
    c = lax.axis_index("c")
    s = lax.axis_index("s")
    _fill_f32(zero_v, ZR, H, 0.0)
    for t in range(RPT // ZR):
        pltpu.sync_copy(zero_v, acc.at[pl.ds(s * RPT + t * ZR, ZR), :])
    plsc.subcore_barrier()

    base = (c * NTILE + s) * EPW

    def body(i, carry):
        e0 = base + i * K
        pltpu.sync_copy(col_hbm.at[pl.ds(e0, K)], col_v)
        pltpu.sync_copy(row_hbm.at[pl.ds(e0, K)], row_v)
        pltpu.async_copy(hp_hbm.at[col_v], rows_v, sem).wait()
        pltpu.sync_copy(rows_v, acc.at[row_v], add=True)
        return carry

    lax.fori_loop(0, NCHUNK, body, 0)
    plsc.subcore_barrier()
    pltpu.sync_copy(
        acc.at[pl.ds(s * RPT, RPT), :],
        out_hbm.at[pl.ds(c * NP + s * RPT, RPT), :],
    )


def _bn(x, g, b):
    mu = jnp.mean(x, axis=0, keepdims=True)
    xc = x - mu
    var = jnp.mean(xc * xc, axis=0, keepdims=True)
    return xc * lax.rsqrt(var + 1e-5) * g + b


def _dis(dp):
    cnt = dp[0, :N] + dp[1, :N]
    return lax.rsqrt(cnt[:, :1] + 1.0)


def _tck_feat(x_ref, g_ref, b_ref, Wf_ref, bf_ref, o_ref):
    h = _bn(x_ref[...], g_ref[...], b_ref[...])
    h = jnp.dot(h, Wf_ref[...], preferred_element_type=jnp.float32) + bf_ref[...]
    o_ref[...] = jnp.maximum(h, 0.0)


def _tck_in(h_ref, dp_ref, g_ref, b_ref, W_ref, o_ref):
    dis = _dis(dp_ref[...])
    u = _bn(h_ref[...], g_ref[...], b_ref[...])
    o_ref[...] = dis * jnp.dot(u, W_ref[...], preferred_element_type=jnp.float32)


def _tck_mid(A_ref, hp_ref, dp_ref, bprev_ref, g_ref, b_ref, W_ref, o_ref):
    dis = _dis(dp_ref[...])
    agg = A_ref[0, :N] + A_ref[1, :N] + hp_ref[...]
    hout = jnp.maximum(dis * agg + bprev_ref[...], 0.0)
    u = _bn(hout, g_ref[...], b_ref[...])
    o_ref[...] = dis * jnp.dot(u, W_ref[...], preferred_element_type=jnp.float32)


def _tck_head(A_ref, hp_ref, dp_ref, b3_ref, batch_ref, fg_ref, fb_ref,
              Wl_ref, bl_ref, hg_ref, hb_ref, Wc_ref, bc_ref, o_ref):
    dis = _dis(dp_ref[...])
    agg = A_ref[0, :N] + A_ref[1, :N] + hp_ref[...]
    h3 = jnp.maximum(dis * agg + b3_ref[...], 0.0)
    b_row = batch_ref[0:1, :]
    seg_ids = lax.broadcasted_iota(jnp.int32, (G, 1), 0)
    mask = (b_row == seg_ids).astype(jnp.float32)
    p = jnp.dot(mask, h3, preferred_element_type=jnp.float32)
    p = _bn(p, fg_ref[...], fb_ref[...])
    p = jnp.maximum(jnp.dot(p, Wl_ref[...], preferred_element_type=jnp.float32) + bl_ref[...], 0.0)
    p = _bn(p, hg_ref[...], hb_ref[...])
    logits = jnp.dot(p, Wc_ref[...], preferred_element_type=jnp.float32) + bc_ref[...]
    m = jnp.max(logits, axis=-1, keepdims=True)
    z = logits - m
    o_ref[...] = z - jnp.log(jnp.sum(jnp.exp(z), axis=-1, keepdims=True))


def _tc(body, out_shape):
    return pl.pallas_call(body, out_shape=jax.ShapeDtypeStruct(out_shape, jnp.float32))


def kernel(x, edge_index, batch, bn_feat_g, bn_feat_b, Wf, bf,
           bn1_g, bn1_b, W1, b1, bn2_g, bn2_b, W2, b2, bn3_g, bn3_b, W3, b3,
           bn_fc_g, bn_fc_b, Wl, bl, bn_h_g, bn_h_b, Wc, bc):
    row = edge_index[0]
    col = edge_index[1]
    r2 = lambda v: v.reshape(1, -1)

    degpair = _sc_degree(row).reshape(NSC, NP, DEGW)
    h0 = _tc(_tck_feat, (N, H))(x, r2(bn_feat_g), r2(bn_feat_b), Wf, r2(bf))
    hp1 = _tc(_tck_in, (N, H))(h0, degpair, r2(bn1_g), r2(bn1_b), W1)
    A1 = _sc_aggregate(hp1, row, col).reshape(NSC, NP, H)
    hp2 = _tc(_tck_mid, (N, H))(A1, hp1, degpair, r2(b1), r2(bn2_g), r2(bn2_b), W2)
    A2 = _sc_aggregate(hp2, row, col).reshape(NSC, NP, H)
    hp3 = _tc(_tck_mid, (N, H))(A2, hp2, degpair, r2(b2), r2(bn3_g), r2(bn3_b), W3)
    A3 = _sc_aggregate(hp3, row, col).reshape(NSC, NP, H)
    batch2d = jnp.broadcast_to(batch[None, :], (8, N))
    out = _tc(_tck_head, (G, C))(
        A3, hp3, degpair, r2(b3), batch2d, r2(bn_fc_g), r2(bn_fc_b),
        Wl, r2(bl), r2(bn_h_g), r2(bn_h_b), Wc, r2(bc))
    return out


# speedup vs baseline: 22.8282x; 2.2327x over previous
import functools

import jax
import jax.numpy as jnp
from jax import lax
from jax.experimental import pallas as pl
from jax.experimental.pallas import tpu as pltpu
from jax.experimental.pallas import tpu_sc as plsc

N = 10000
E = 320000
D = 128
H = 128
C = 10
G = 64

NSC = 2
NTILE = 16
NW = NSC * NTILE
EPW = E // NW
NP = 10240
RPT = NP // NTILE
K = 80
NCHUNK = EPW // K
DEGW = 128

_MESH = plsc.VectorSubcoreMesh(core_axis_name="c", subcore_axis_name="s")


def _fill_f32(ref, rows, width, value):
    vec = jnp.full((16,), value, jnp.float32)

    def body(r, carry):
        for j in range(width // 16):
            ref[r, pl.ds(j * 16, 16)] = vec
        return carry

    lax.fori_loop(0, rows, body, 0)


@functools.partial(
    pl.kernel,
    out_type=jax.ShapeDtypeStruct((NSC * NP, DEGW), jnp.float32),
    mesh=_MESH,
    scratch_types=[
        pltpu.VMEM((K, DEGW), jnp.float32),
        pltpu.VMEM((NCHUNK, K), jnp.int32),
        pltpu.VMEM_SHARED((NP, DEGW), jnp.float32),
        pltpu.SemaphoreType.DMA,
    ],
)
def _sc_degree(row_hbm, out_hbm, ones_v, idx_v, acc, sem):
    c = lax.axis_index("c")
    s = lax.axis_index("s")
    w = c * NTILE + s
    pltpu.sync_copy(row_hbm.at[w], idx_v)
    _fill_f32(ones_v, K, DEGW, 0.0)
    for t in range(RPT // K):
        pltpu.sync_copy(ones_v, acc.at[pl.ds(s * RPT + t * K, K), :])
    _fill_f32(ones_v, K, DEGW, 1.0)
    plsc.subcore_barrier()

    FK = 5

    def body(j, carry):
        for t in range(FK):
            pltpu.async_copy(ones_v, acc.at[idx_v.at[j * FK + t]], sem, add=True)
        for t in range(FK):
            pltpu.make_async_copy(ones_v, acc.at[idx_v.at[0]], sem).wait()
        return carry

    lax.fori_loop(0, NCHUNK // FK, body, 0)
    plsc.subcore_barrier()
    pltpu.sync_copy(
        acc.at[pl.ds(s * RPT, RPT), :],
        out_hbm.at[pl.ds(c * NP + s * RPT, RPT), :],
    )


@functools.partial(
    pl.kernel,
    out_type=jax.ShapeDtypeStruct((NSC * NP, H), jnp.float32),
    mesh=_MESH,
    scratch_types=[
        pltpu.VMEM((EPW,), jnp.int32),
        pltpu.VMEM((NCHUNK, K), jnp.int32),
        pltpu.VMEM((K, H), jnp.float32),
        pltpu.VMEM((K, H), jnp.float32),
        pltpu.VMEM_SHARED((NP, H), jnp.float32),
        pltpu.SemaphoreType.DMA,
        pltpu.SemaphoreType.DMA,
    ],
)
def _sc_aggregate(hp_hbm, row_hbm, col_hbm, out_hbm, col_v, row_v, b0, b1, acc, sem0, sem1):
    c = lax.axis_index("c")
    s = lax.axis_index("s")
    w = c * NTILE + s
    pltpu.sync_copy(col_hbm.at[pl.ds(w * EPW, EPW)], col_v)
    pltpu.sync_copy(row_hbm.at[w], row_v)
    _fill_f32(b0, K, H, 0.0)
    for t in range(RPT // K):
        pltpu.sync_copy(b0, acc.at[pl.ds(s * RPT + t * K, K), :])
    plsc.subcore_barrier()

    def gather(chunk, buf, sem):
        pltpu.async_copy(hp_hbm.at[col_v.at[pl.ds(chunk * K, K)]], buf, sem)

    def drain(buf, sem):
        pltpu.make_async_copy(hp_hbm.at[col_v.at[pl.ds(0, K)]], buf, sem).wait()

    gather(0, b0, sem0)

    def pair(j, carry):
        c0 = 2 * j
        gather(c0 + 1, b1, sem1)
        drain(b0, sem0)
        pltpu.sync_copy(b0, acc.at[row_v.at[c0]], add=True)
        gather(c0 + 2, b0, sem0)
        drain(b1, sem1)
        pltpu.sync_copy(b1, acc.at[row_v.at[c0 + 1]], add=True)
        return carry

    lax.fori_loop(0, (NCHUNK - 1) // 2, pair, 0)
    drain(b0, sem0)
    pltpu.sync_copy(b0, acc.at[row_v.at[NCHUNK - 1]], add=True)
    plsc.subcore_barrier()
    pltpu.sync_copy(
        acc.at[pl.ds(s * RPT, RPT), :],
        out_hbm.at[pl.ds(c * NP + s * RPT, RPT), :],
    )


def _bn(x, g, b):
    mu = jnp.mean(x, axis=0, keepdims=True)
    xc = x - mu
    var = jnp.mean(xc * xc, axis=0, keepdims=True)
    return xc * lax.rsqrt(var + 1e-5) * g + b


def _dis(dp):
    cnt = dp[0, :N] + dp[1, :N]
    return lax.rsqrt(cnt[:, :1] + 1.0)


def _tck_feat(x_ref, g_ref, b_ref, Wf_ref, bf_ref, o_ref):
    h = _bn(x_ref[...], g_ref[...], b_ref[...])
    h = jnp.dot(h, Wf_ref[...], preferred_element_type=jnp.float32) + bf_ref[...]
    o_ref[...] = jnp.maximum(h, 0.0)


def _tck_in(h_ref, dp_ref, g_ref, b_ref, W_ref, o_ref):
    dis = _dis(dp_ref[...])
    u = _bn(h_ref[...], g_ref[...], b_ref[...])
    o_ref[...] = dis * jnp.dot(u, W_ref[...], preferred_element_type=jnp.float32)


def _tck_mid(A_ref, hp_ref, dp_ref, bprev_ref, g_ref, b_ref, W_ref, o_ref):
    dis = _dis(dp_ref[...])
    agg = A_ref[0, :N] + A_ref[1, :N] + hp_ref[...]
    hout = jnp.maximum(dis * agg + bprev_ref[...], 0.0)
    u = _bn(hout, g_ref[...], b_ref[...])
    o_ref[...] = dis * jnp.dot(u, W_ref[...], preferred_element_type=jnp.float32)


def _tck_head(A_ref, hp_ref, dp_ref, b3_ref, batch_ref, fg_ref, fb_ref,
              Wl_ref, bl_ref, hg_ref, hb_ref, Wc_ref, bc_ref, o_ref):
    dis = _dis(dp_ref[...])
    agg = A_ref[0, :N] + A_ref[1, :N] + hp_ref[...]
    h3 = jnp.maximum(dis * agg + b3_ref[...], 0.0)
    b_row = batch_ref[0:1, :]
    seg_ids = lax.broadcasted_iota(jnp.int32, (G, 1), 0)
    mask = (b_row == seg_ids).astype(jnp.float32)
    p = jnp.dot(mask, h3, preferred_element_type=jnp.float32)
    p = _bn(p, fg_ref[...], fb_ref[...])
    p = jnp.maximum(jnp.dot(p, Wl_ref[...], preferred_element_type=jnp.float32) + bl_ref[...], 0.0)
    p = _bn(p, hg_ref[...], hb_ref[...])
    logits = jnp.dot(p, Wc_ref[...], preferred_element_type=jnp.float32) + bc_ref[...]
    m = jnp.max(logits, axis=-1, keepdims=True)
    z = logits - m
    o_ref[...] = z - jnp.log(jnp.sum(jnp.exp(z), axis=-1, keepdims=True))


def _tc(body, out_shape):
    return pl.pallas_call(body, out_shape=jax.ShapeDtypeStruct(out_shape, jnp.float32))


def kernel(x, edge_index, batch, bn_feat_g, bn_feat_b, Wf, bf,
           bn1_g, bn1_b, W1, b1, bn2_g, bn2_b, W2, b2, bn3_g, bn3_b, W3, b3,
           bn_fc_g, bn_fc_b, Wl, bl, bn_h_g, bn_h_b, Wc, bc):
    row = edge_index[0].reshape(NW, NCHUNK, K)
    col = edge_index[1]
    r2 = lambda v: v.reshape(1, -1)

    degpair = _sc_degree(row).reshape(NSC, NP, DEGW)
    h0 = _tc(_tck_feat, (N, H))(x, r2(bn_feat_g), r2(bn_feat_b), Wf, r2(bf))
    hp1 = _tc(_tck_in, (N, H))(h0, degpair, r2(bn1_g), r2(bn1_b), W1)
    A1 = _sc_aggregate(hp1, row, col).reshape(NSC, NP, H)
    hp2 = _tc(_tck_mid, (N, H))(A1, hp1, degpair, r2(b1), r2(bn2_g), r2(bn2_b), W2)
    A2 = _sc_aggregate(hp2, row, col).reshape(NSC, NP, H)
    hp3 = _tc(_tck_mid, (N, H))(A2, hp2, degpair, r2(b2), r2(bn3_g), r2(bn3_b), W3)
    A3 = _sc_aggregate(hp3, row, col).reshape(NSC, NP, H)
    batch2d = jnp.broadcast_to(batch[None, :], (8, N))
    out = _tc(_tck_head, (G, C))(
        A3, hp3, degpair, r2(b3), batch2d, r2(bn_fc_g), r2(bn_fc_b),
        Wl, r2(bl), r2(bn_h_g), r2(bn_h_b), Wc, r2(bc))
    return out
